# baseline (speedup 1.0000x reference)
import functools

import jax
import jax.numpy as jnp
from jax import lax
from jax.experimental import pallas as pl
from jax.experimental.pallas import tpu as pltpu
from jax.experimental.pallas import tpu_sc as plsc

E = 320000
N = 10000
D = 128
R = 16
T = 12

_SLICES = (87040, 81920, 76800, 74240)
_NSLICE = len(_SLICES)
_SLICE_OFF = tuple(sum(_SLICES[:i]) for i in range(_NSLICE))
assert sum(_SLICES) == E

_EB = 1280


def _edge_body(xl_ref, xh_ref, rl_ref, rh_ref, w_ref, out_ref):
    def half(rbf_t_ref, x_ref):
        g = lax.dot_general(rbf_t_ref[...], w_ref[...],
                            (((0,), (0,)), ((), ())),
                            preferred_element_type=jnp.float32)
        m = (g * x_ref[...]).astype(jnp.bfloat16)
        a = lax.bitcast_convert_type(m[:, :64], jnp.uint16)
        b = lax.bitcast_convert_type(m[:, 64:], jnp.uint16)
        word = a.astype(jnp.uint32) | (b.astype(jnp.uint32) << 16)
        return lax.bitcast_convert_type(word, jnp.int32)

    out_ref[...] = jnp.concatenate(
        [half(rl_ref, xl_ref), half(rh_ref, xh_ref)], axis=1)


def _edge_stage(x, rbf, W_rbf, sl):
    es = _SLICES[sl]
    bps = es // (2 * _EB)
    lo = _SLICE_OFF[sl] // _EB
    hi = lo + bps
    return pl.pallas_call(
        _edge_body,
        grid=(bps,),
        in_specs=[
            pl.BlockSpec((_EB, D), lambda i: (i + lo, 0)),
            pl.BlockSpec((_EB, D), lambda i: (i + hi, 0)),
            pl.BlockSpec((R, _EB), lambda i: (0, i + lo)),
            pl.BlockSpec((R, _EB), lambda i: (0, i + hi)),
            pl.BlockSpec((R, D), lambda i: (0, 0)),
        ],
        out_specs=pl.BlockSpec((_EB, D), lambda i: (i, 0)),
        out_shape=jax.ShapeDtypeStruct((es // 2, D), jnp.int32),
    )(x, x, rbf.T, rbf.T, W_rbf)


_NC = 2
_NS = 16
_NW = _NC * _NS
_CH = 64
N_PAD = 10112
_RPW = N_PAD // _NS

_sc_mesh = plsc.VectorSubcoreMesh(core_axis_name="c", subcore_axis_name="s")


def _make_scatter(sl, chained):
    es = _SLICES[sl]
    rw = es // 2 // _NW
    cf = rw // _CH
    tail = rw - cf * _CH
    idx_base0 = _SLICE_OFF[sl]
    ic = _NC if chained else 1
    assert cf >= 4 and rw % 8 == 0 and tail % 8 == 0

    scratch = [
        pltpu.VMEM_SHARED((N_PAD, D), jnp.float32),
        pltpu.SemaphoreType.DMA,
        pltpu.SemaphoreType.DMA,
        pltpu.SemaphoreType.DMA,
        pltpu.SemaphoreType.DMA,
        pltpu.SemaphoreType.DMA,
        pltpu.SemaphoreType.DMA,
        pltpu.SemaphoreType.DMA,
        pltpu.SemaphoreType.DMA,
    ]

    def _impl(h_hbm, idx_hbm, init_hbm, out_hbm, acc,
              sw0, sw1, sil0, sih0, sil1, sih1, ss0, ss1):
      def _body(hb0, hb1, f0, f1, i0, i1, j0, j1, it_):
        c = lax.axis_index("c")
        s = lax.axis_index("s")
        rbase = (c * _NS + s) * rw
        ibase_lo = idx_base0 + rbase
        ibase_hi = idx_base0 + es // 2 + rbase
        HB, F, I, J = (hb0, hb1), (f0, f1), (i0, i1), (j0, j1)
        SW, SIL, SIH, SS = (sw0, sw1), (sil0, sil1), (sih0, sih1), (ss0, ss1)

        if chained:
            pltpu.sync_copy(init_hbm.at[c, pl.ds(s * _RPW, _RPW)],
                            acc.at[pl.ds(s * _RPW, _RPW)])
        else:
            zf = jnp.zeros((16,), jnp.float32)

            @plsc.parallel_loop(0, 2 * _CH, step=1, unroll=8)
            def _zrow(r):
                for gg in range(8):
                    f0[r, pl.ds(gg * 16, 16)] = zf

            nfull = _RPW // (2 * _CH)
            rem = _RPW - nfull * 2 * _CH
            for rep in range(nfull):
                pltpu.sync_copy(
                    f0, acc.at[pl.ds(s * _RPW + rep * 2 * _CH, 2 * _CH)])
            if rem:
                pltpu.sync_copy(
                    f0.at[pl.ds(0, rem)],
                    acc.at[pl.ds(s * _RPW + nfull * 2 * _CH, rem)])
        plsc.subcore_barrier()

        def load(k, b):
            pltpu.async_copy(h_hbm.at[pl.ds(rbase + k * _CH, _CH)],
                             HB[b], SW[b])
            pltpu.async_copy(idx_hbm.at[pl.ds(ibase_lo + k * _CH, _CH)],
                             I[b].at[pl.ds(0, _CH)], SIL[b])
            pltpu.async_copy(idx_hbm.at[pl.ds(ibase_hi + k * _CH, _CH)],
                             I[b].at[pl.ds(_CH, _CH)], SIH[b])

        def wload(b):
            pltpu.make_async_copy(h_hbm.at[pl.ds(0, _CH)], HB[b],
                                  SW[b]).wait()
            pltpu.make_async_copy(idx_hbm.at[pl.ds(0, _CH)],
                                  I[b].at[pl.ds(0, _CH)], SIL[b]).wait()
            pltpu.make_async_copy(idx_hbm.at[pl.ds(0, _CH)],
                                  I[b].at[pl.ds(0, _CH)], SIH[b]).wait()

        def conv(hb, f, nrows):
            @plsc.parallel_loop(0, nrows, step=1, unroll=4)
            def _row(r):
                for widx, roff in ((0, 0), (64, nrows)):
                    for g in range(4):
                        v = hb[r, pl.ds(widx + g * 16, 16)]
                        f[roff + r, pl.ds(g * 16, 16)] = \
                            lax.bitcast_convert_type(v << 16, jnp.float32)
                        f[roff + r, pl.ds(64 + g * 16, 16)] = \
                            lax.bitcast_convert_type(
                                v & jnp.int32(-65536), jnp.float32)

        def wait_scat(b):
            pltpu.make_async_copy(F[b], acc.at[J[b]], SS[b]).wait()

        def proc(b, wait_prev=True):
            wload(b)
            for g in range(2 * _CH // 16):
                J[b][pl.ds(g * 16, 16)] = I[b][pl.ds(g * 16, 16)]
            conv(HB[b], F[b], _CH)
            if wait_prev:
                wait_scat(1 - b)
            pltpu.async_copy(F[b], acc.at[J[b]], SS[b], add=True)

        load(0, 0)
        load(1, 1)
        proc(0, wait_prev=False)
        load(2, 0)

        np_steady = (cf - 3) // 2

        def pair(t, carry):
            proc(1)
            load(2 * t + 3, 1)
            proc(0)
            load(2 * t + 4, 0)
            return carry

        lax.fori_loop(0, np_steady, pair, 0)

        loaded = 2 * np_steady + 2
        for k in range(2 * np_steady + 1, cf):
            proc(k % 2)
            nxt = k + 2
            if nxt < cf and nxt > loaded:
                load(nxt, nxt % 2)
                loaded = nxt
        wait_scat((cf - 1) % 2)

        if tail:
            toff = cf * _CH
            pltpu.sync_copy(h_hbm.at[pl.ds(rbase + toff, tail)],
                            hb0.at[pl.ds(0, tail)])
            pltpu.sync_copy(idx_hbm.at[pl.ds(ibase_lo + toff, tail)],
                            it_.at[pl.ds(0, tail)])
            pltpu.sync_copy(idx_hbm.at[pl.ds(ibase_hi + toff, tail)],
                            it_.at[pl.ds(tail, tail)])
            conv(hb0, f0, tail)
            pltpu.sync_copy(f0.at[pl.ds(0, 2 * tail)], acc.at[it_], add=True)

        plsc.subcore_barrier()
        pltpu.sync_copy(acc.at[pl.ds(s * _RPW, _RPW)],
                        out_hbm.at[c, pl.ds(s * _RPW, _RPW)])

      pl.run_scoped(
          _body,
          pltpu.VMEM((_CH, D), jnp.int32),
          pltpu.VMEM((_CH, D), jnp.int32),
          pltpu.VMEM((2 * _CH, D), jnp.float32),
          pltpu.VMEM((2 * _CH, D), jnp.float32),
          pltpu.VMEM((2 * _CH,), jnp.int32),
          pltpu.VMEM((2 * _CH,), jnp.int32),
          pltpu.VMEM((2 * _CH,), jnp.int32),
          pltpu.VMEM((2 * _CH,), jnp.int32),
          pltpu.VMEM((2 * max(tail, 8),), jnp.int32),
      )

    kw = dict(out_type=jax.ShapeDtypeStruct((_NC, N_PAD, D), jnp.float32),
              mesh=_sc_mesh, scratch_types=scratch)
    if chained:
        @functools.partial(pl.kernel, **kw)
        def _scatter_stage(h_hbm, idx_hbm, init_hbm, out_hbm, acc,
                           sw0, sw1, sil0, sih0, sil1, sih1, ss0, ss1):
            _impl(h_hbm, idx_hbm, init_hbm, out_hbm, acc,
                  sw0, sw1, sil0, sih0, sil1, sih1, ss0, ss1)
    else:
        @functools.partial(pl.kernel, **kw)
        def _scatter_stage(h_hbm, idx_hbm, out_hbm, acc,
                           sw0, sw1, sil0, sih0, sil1, sih1, ss0, ss1):
            _impl(h_hbm, idx_hbm, None, out_hbm, acc,
                  sw0, sw1, sil0, sih0, sil1, sih1, ss0, ss1)

    return _scatter_stage


_scatter_stages = [_make_scatter(sl, sl > 0) for sl in range(_NSLICE)]

_NODE_BLK = 2528


def _mlp_body(p_ref, w1_ref, b1_ref, w2_ref, b2_ref, w3_ref, b3_ref, wo_ref,
              out_ref):
    h = p_ref[0] + p_ref[1]
    h = jax.nn.silu(jnp.dot(h, w1_ref[...],
                            preferred_element_type=jnp.float32) + b1_ref[...])
    h = jax.nn.silu(jnp.dot(h, w2_ref[...],
                            preferred_element_type=jnp.float32) + b2_ref[...])
    h = jax.nn.silu(jnp.dot(h, w3_ref[...],
                            preferred_element_type=jnp.float32) + b3_ref[...])
    out_ref[...] = jnp.dot(h, wo_ref[...], preferred_element_type=jnp.float32)


def _mlp_stage(partials, W1, b1, W2, b2, W3, b3, W_out):
    grid = (N_PAD // _NODE_BLK,)
    full = lambda i: (0, 0)
    return pl.pallas_call(
        _mlp_body,
        grid=grid,
        in_specs=[
            pl.BlockSpec((_NC, _NODE_BLK, D), lambda i: (0, i, 0)),
            pl.BlockSpec((D, D), full),
            pl.BlockSpec((1, D), full),
            pl.BlockSpec((D, D), full),
            pl.BlockSpec((1, D), full),
            pl.BlockSpec((D, D), full),
            pl.BlockSpec((1, D), full),
            pl.BlockSpec((D, T), full),
        ],
        out_specs=pl.BlockSpec((_NODE_BLK, T), lambda i: (i, 0)),
        out_shape=jax.ShapeDtypeStruct((N, T), jnp.float32),
    )(partials, W1, b1, W2, b2, W3, b3, W_out)


def kernel(x, rbf, idnb_i, n_atoms, W_rbf, W1, b1, W2, b2, W3, b3, W_out):
    del n_atoms
    p = None
    for sl in range(_NSLICE):
        h_s = _edge_stage(x, rbf, W_rbf, sl)
        args = (h_s, idnb_i) if p is None else (h_s, idnb_i, p)
        p = _scatter_stages[sl](*args)
    return _mlp_stage(p, W1, b1.reshape(1, D), W2, b2.reshape(1, D),
                      W3, b3.reshape(1, D), W_out)


# speedup vs baseline: 1.0081x; 1.0081x over previous
import functools

import jax
import jax.numpy as jnp
from jax import lax
from jax.experimental import pallas as pl
from jax.experimental.pallas import tpu as pltpu
from jax.experimental.pallas import tpu_sc as plsc

E = 320000
N = 10000
D = 128
R = 16
T = 12

_SLICES = (125440, 104960, 89600)
_NSLICE = len(_SLICES)
_SLICE_OFF = tuple(sum(_SLICES[:i]) for i in range(_NSLICE))
assert sum(_SLICES) == E

_EB = 1280


def _edge_body(xl_ref, xh_ref, rl_ref, rh_ref, w_ref, out_ref):
    def half(rbf_t_ref, x_ref):
        g = lax.dot_general(rbf_t_ref[...], w_ref[...],
                            (((0,), (0,)), ((), ())),
                            preferred_element_type=jnp.float32)
        m = (g * x_ref[...]).astype(jnp.bfloat16)
        a = lax.bitcast_convert_type(m[:, :64], jnp.uint16)
        b = lax.bitcast_convert_type(m[:, 64:], jnp.uint16)
        word = a.astype(jnp.uint32) | (b.astype(jnp.uint32) << 16)
        return lax.bitcast_convert_type(word, jnp.int32)

    out_ref[...] = jnp.concatenate(
        [half(rl_ref, xl_ref), half(rh_ref, xh_ref)], axis=1)


def _edge_stage(x, rbf, W_rbf, sl):
    es = _SLICES[sl]
    bps = es // (2 * _EB)
    lo = _SLICE_OFF[sl] // _EB
    hi = lo + bps
    return pl.pallas_call(
        _edge_body,
        grid=(bps,),
        in_specs=[
            pl.BlockSpec((_EB, D), lambda i: (i + lo, 0)),
            pl.BlockSpec((_EB, D), lambda i: (i + hi, 0)),
            pl.BlockSpec((R, _EB), lambda i: (0, i + lo)),
            pl.BlockSpec((R, _EB), lambda i: (0, i + hi)),
            pl.BlockSpec((R, D), lambda i: (0, 0)),
        ],
        out_specs=pl.BlockSpec((_EB, D), lambda i: (i, 0)),
        out_shape=jax.ShapeDtypeStruct((es // 2, D), jnp.int32),
    )(x, x, rbf.T, rbf.T, W_rbf)


_NC = 2
_NS = 16
_NW = _NC * _NS
_CH = 64
N_PAD = 10112
_RPW = N_PAD // _NS

_sc_mesh = plsc.VectorSubcoreMesh(core_axis_name="c", subcore_axis_name="s")


def _make_scatter(sl, chained):
    es = _SLICES[sl]
    rw = es // 2 // _NW
    cf = rw // _CH
    tail = rw - cf * _CH
    idx_base0 = _SLICE_OFF[sl]
    ic = _NC if chained else 1
    assert cf >= 4 and rw % 8 == 0 and tail % 8 == 0

    scratch = [
        pltpu.VMEM_SHARED((N_PAD, D), jnp.float32),
        pltpu.SemaphoreType.DMA,
        pltpu.SemaphoreType.DMA,
        pltpu.SemaphoreType.DMA,
        pltpu.SemaphoreType.DMA,
        pltpu.SemaphoreType.DMA,
        pltpu.SemaphoreType.DMA,
        pltpu.SemaphoreType.DMA,
        pltpu.SemaphoreType.DMA,
    ]

    def _impl(h_hbm, idx_hbm, init_hbm, out_hbm, acc,
              sw0, sw1, sil0, sih0, sil1, sih1, ss0, ss1):
      def _body(hb0, hb1, f0, f1, i0, i1, j0, j1, it_):
        c = lax.axis_index("c")
        s = lax.axis_index("s")
        rbase = (c * _NS + s) * rw
        ibase_lo = idx_base0 + rbase
        ibase_hi = idx_base0 + es // 2 + rbase
        HB, F, I, J = (hb0, hb1), (f0, f1), (i0, i1), (j0, j1)
        SW, SIL, SIH, SS = (sw0, sw1), (sil0, sil1), (sih0, sih1), (ss0, ss1)

        if chained:
            pltpu.sync_copy(init_hbm.at[c, pl.ds(s * _RPW, _RPW)],
                            acc.at[pl.ds(s * _RPW, _RPW)])
        else:
            zf = jnp.zeros((16,), jnp.float32)

            @plsc.parallel_loop(0, 2 * _CH, step=1, unroll=8)
            def _zrow(r):
                for gg in range(8):
                    f0[r, pl.ds(gg * 16, 16)] = zf

            nfull = _RPW // (2 * _CH)
            rem = _RPW - nfull * 2 * _CH
            for rep in range(nfull):
                pltpu.sync_copy(
                    f0, acc.at[pl.ds(s * _RPW + rep * 2 * _CH, 2 * _CH)])
            if rem:
                pltpu.sync_copy(
                    f0.at[pl.ds(0, rem)],
                    acc.at[pl.ds(s * _RPW + nfull * 2 * _CH, rem)])
        plsc.subcore_barrier()

        def load(k, b):
            pltpu.async_copy(h_hbm.at[pl.ds(rbase + k * _CH, _CH)],
                             HB[b], SW[b])
            pltpu.async_copy(idx_hbm.at[pl.ds(ibase_lo + k * _CH, _CH)],
                             I[b].at[pl.ds(0, _CH)], SIL[b])
            pltpu.async_copy(idx_hbm.at[pl.ds(ibase_hi + k * _CH, _CH)],
                             I[b].at[pl.ds(_CH, _CH)], SIH[b])

        def wload(b):
            pltpu.make_async_copy(h_hbm.at[pl.ds(0, _CH)], HB[b],
                                  SW[b]).wait()
            pltpu.make_async_copy(idx_hbm.at[pl.ds(0, _CH)],
                                  I[b].at[pl.ds(0, _CH)], SIL[b]).wait()
            pltpu.make_async_copy(idx_hbm.at[pl.ds(0, _CH)],
                                  I[b].at[pl.ds(0, _CH)], SIH[b]).wait()

        def conv(hb, f, nrows):
            @plsc.parallel_loop(0, nrows, step=1, unroll=4)
            def _row(r):
                for widx, roff in ((0, 0), (64, nrows)):
                    for g in range(4):
                        v = hb[r, pl.ds(widx + g * 16, 16)]
                        f[roff + r, pl.ds(g * 16, 16)] = \
                            lax.bitcast_convert_type(v << 16, jnp.float32)
                        f[roff + r, pl.ds(64 + g * 16, 16)] = \
                            lax.bitcast_convert_type(
                                v & jnp.int32(-65536), jnp.float32)

        def wait_scat(b):
            pltpu.make_async_copy(F[b], acc.at[J[b]], SS[b]).wait()

        def proc(b, wait_prev=True):
            wload(b)
            for g in range(2 * _CH // 16):
                J[b][pl.ds(g * 16, 16)] = I[b][pl.ds(g * 16, 16)]
            conv(HB[b], F[b], _CH)
            if wait_prev:
                wait_scat(1 - b)
            pltpu.async_copy(F[b], acc.at[J[b]], SS[b], add=True)

        load(0, 0)
        load(1, 1)
        proc(0, wait_prev=False)
        load(2, 0)

        np_steady = (cf - 3) // 2

        def pair(t, carry):
            proc(1)
            load(2 * t + 3, 1)
            proc(0)
            load(2 * t + 4, 0)
            return carry

        lax.fori_loop(0, np_steady, pair, 0)

        loaded = 2 * np_steady + 2
        for k in range(2 * np_steady + 1, cf):
            proc(k % 2)
            nxt = k + 2
            if nxt < cf and nxt > loaded:
                load(nxt, nxt % 2)
                loaded = nxt
        wait_scat((cf - 1) % 2)

        if tail:
            toff = cf * _CH
            pltpu.sync_copy(h_hbm.at[pl.ds(rbase + toff, tail)],
                            hb0.at[pl.ds(0, tail)])
            pltpu.sync_copy(idx_hbm.at[pl.ds(ibase_lo + toff, tail)],
                            it_.at[pl.ds(0, tail)])
            pltpu.sync_copy(idx_hbm.at[pl.ds(ibase_hi + toff, tail)],
                            it_.at[pl.ds(tail, tail)])
            conv(hb0, f0, tail)
            pltpu.sync_copy(f0.at[pl.ds(0, 2 * tail)], acc.at[it_], add=True)

        plsc.subcore_barrier()
        pltpu.sync_copy(acc.at[pl.ds(s * _RPW, _RPW)],
                        out_hbm.at[c, pl.ds(s * _RPW, _RPW)])

      pl.run_scoped(
          _body,
          pltpu.VMEM((_CH, D), jnp.int32),
          pltpu.VMEM((_CH, D), jnp.int32),
          pltpu.VMEM((2 * _CH, D), jnp.float32),
          pltpu.VMEM((2 * _CH, D), jnp.float32),
          pltpu.VMEM((2 * _CH,), jnp.int32),
          pltpu.VMEM((2 * _CH,), jnp.int32),
          pltpu.VMEM((2 * _CH,), jnp.int32),
          pltpu.VMEM((2 * _CH,), jnp.int32),
          pltpu.VMEM((2 * max(tail, 8),), jnp.int32),
      )

    kw = dict(out_type=jax.ShapeDtypeStruct((_NC, N_PAD, D), jnp.float32),
              mesh=_sc_mesh, scratch_types=scratch)
    if chained:
        @functools.partial(pl.kernel, **kw)
        def _scatter_stage(h_hbm, idx_hbm, init_hbm, out_hbm, acc,
                           sw0, sw1, sil0, sih0, sil1, sih1, ss0, ss1):
            _impl(h_hbm, idx_hbm, init_hbm, out_hbm, acc,
                  sw0, sw1, sil0, sih0, sil1, sih1, ss0, ss1)
    else:
        @functools.partial(pl.kernel, **kw)
        def _scatter_stage(h_hbm, idx_hbm, out_hbm, acc,
                           sw0, sw1, sil0, sih0, sil1, sih1, ss0, ss1):
            _impl(h_hbm, idx_hbm, None, out_hbm, acc,
                  sw0, sw1, sil0, sih0, sil1, sih1, ss0, ss1)

    return _scatter_stage


_scatter_stages = [_make_scatter(sl, sl > 0) for sl in range(_NSLICE)]

_NODE_BLK = 2528


def _mlp_body(p_ref, w1_ref, b1_ref, w2_ref, b2_ref, w3_ref, b3_ref, wo_ref,
              out_ref):
    h = p_ref[0] + p_ref[1]
    h = jax.nn.silu(jnp.dot(h, w1_ref[...],
                            preferred_element_type=jnp.float32) + b1_ref[...])
    h = jax.nn.silu(jnp.dot(h, w2_ref[...],
                            preferred_element_type=jnp.float32) + b2_ref[...])
    h = jax.nn.silu(jnp.dot(h, w3_ref[...],
                            preferred_element_type=jnp.float32) + b3_ref[...])
    out_ref[...] = jnp.dot(h, wo_ref[...], preferred_element_type=jnp.float32)


def _mlp_stage(partials, W1, b1, W2, b2, W3, b3, W_out):
    grid = (N_PAD // _NODE_BLK,)
    full = lambda i: (0, 0)
    return pl.pallas_call(
        _mlp_body,
        grid=grid,
        in_specs=[
            pl.BlockSpec((_NC, _NODE_BLK, D), lambda i: (0, i, 0)),
            pl.BlockSpec((D, D), full),
            pl.BlockSpec((1, D), full),
            pl.BlockSpec((D, D), full),
            pl.BlockSpec((1, D), full),
            pl.BlockSpec((D, D), full),
            pl.BlockSpec((1, D), full),
            pl.BlockSpec((D, T), full),
        ],
        out_specs=pl.BlockSpec((_NODE_BLK, T), lambda i: (i, 0)),
        out_shape=jax.ShapeDtypeStruct((N, T), jnp.float32),
    )(partials, W1, b1, W2, b2, W3, b3, W_out)


def kernel(x, rbf, idnb_i, n_atoms, W_rbf, W1, b1, W2, b2, W3, b3, W_out):
    del n_atoms
    p = None
    for sl in range(_NSLICE):
        h_s = _edge_stage(x, rbf, W_rbf, sl)
        args = (h_s, idnb_i) if p is None else (h_s, idnb_i, p)
        p = _scatter_stages[sl](*args)
    return _mlp_stage(p, W1, b1.reshape(1, D), W2, b2.reshape(1, D),
                      W3, b3.reshape(1, D), W_out)
